# baseline (speedup 1.0000x reference)
import jax
import jax.numpy as jnp
from jax import lax
from jax.experimental import pallas as pl
from jax.experimental.pallas import tpu as pltpu
from jax.experimental.pallas import tpu_sc as plsc

_NOBJ = 64
_V = 262144
_R = (_V * 3) // 128
_NC = 2
_NS = 16
_NW = _NC * _NS
_RW = _R // _NW


def _sc_body(obj_hbm, tbl_hbm, out_hbm, obj_v, buf_v):
    c = lax.axis_index("c")
    s = lax.axis_index("s")
    wid = s * _NC + c
    pltpu.sync_copy(obj_hbm, obj_v)
    obj = obj_v[...][0]
    base = wid * _RW
    pltpu.sync_copy(tbl_hbm.at[obj, pl.ds(base, _RW)], buf_v)
    pltpu.sync_copy(buf_v, out_hbm.at[0, pl.ds(base, _RW)])


_gather = pl.kernel(
    _sc_body,
    out_type=jax.ShapeDtypeStruct((1, _R, 128), jnp.float32),
    mesh=plsc.VectorSubcoreMesh(core_axis_name="c", subcore_axis_name="s"),
    scratch_types=[
        pltpu.VMEM((16,), jnp.int32),
        pltpu.VMEM((_RW, 128), jnp.float32),
    ],
)


def kernel(obj_id, weights):
    obj = jnp.asarray(obj_id, dtype=jnp.int32)
    w = weights.reshape(_NOBJ, _R, 128)
    out_v = _gather(jnp.full((16,), obj, dtype=jnp.int32), w)
    return out_v.reshape(1, _V, 3)
